# baseline (speedup 1.0000x reference)
import functools

import numpy as np
import jax
import jax.numpy as jnp
from jax import lax
from jax.experimental import pallas as pl
from jax.experimental.pallas import tpu as pltpu
from jax.experimental.pallas import tpu_sc as plsc

B = 4096
L = 200
D = 64
DPAD = 128
N_ROWS = B * L
NW = 32
PER_W = N_ROWS // NW
SUBG = 128
CHUNK = 256
BLK = 10
IDX_BLK = BLK * CHUNK
N_CHUNKS = PER_W // CHUNK
N_PAIRS = N_CHUNKS // (2 * BLK)
LANES = 16


def _positional_encoding():
    pos = np.arange(L, dtype=np.float32)[:, None]
    div = np.exp(np.arange(0, D, 2, dtype=np.float32) * (-np.log(10000.0) / D))
    pe = np.zeros((L, D), dtype=np.float32)
    pe[:, 0::2] = np.sin(pos * div)
    pe[:, 1::2] = np.cos(pos * div)
    return jnp.asarray(pe.reshape(-1))


@functools.partial(
    pl.kernel,
    mesh=plsc.VectorSubcoreMesh(core_axis_name="c", subcore_axis_name="s"),
    compiler_params=pltpu.CompilerParams(use_tc_tiling_on_sc=True),
    out_type=jax.ShapeDtypeStruct((N_ROWS, D), jnp.float32),
    scratch_types=[
        pltpu.VMEM((IDX_BLK,), jnp.int32),
        pltpu.VMEM((IDX_BLK,), jnp.int32),
        pltpu.VMEM((CHUNK, DPAD), jnp.float32),
        pltpu.VMEM((CHUNK, DPAD), jnp.float32),
        pltpu.VMEM((CHUNK, D), jnp.float32),
        pltpu.VMEM((L * D,), jnp.float32),
        pltpu.SemaphoreType.DMA,
        pltpu.SemaphoreType.DMA,
        pltpu.SemaphoreType.DMA,
        pltpu.SemaphoreType.DMA,
    ],
)
def _sc_embed(seq_hbm, pe_hbm, table_hbm, out_hbm,
              idx_a, idx_b, wide0, wide1, comp_v, pe_v,
              sg0, sg1, si_a, si_b):
    nc = lax.axis_size("c")
    wid = lax.axis_index("s") * nc + lax.axis_index("c")
    row0 = wid * PER_W
    pltpu.sync_copy(pe_hbm, pe_v)
    pltpu.sync_copy(seq_hbm.at[pl.ds(row0, IDX_BLK)], idx_a)

    def issue_gather(idx_ref, off, wide_ref, sem):
        for j in range(CHUNK // SUBG):
            pltpu.async_copy(
                table_hbm.at[idx_ref.at[pl.ds(off + j * SUBG, SUBG)]],
                wide_ref.at[pl.ds(j * SUBG, SUBG)],
                sem,
            )

    def wait_gather(wide_ref, sem):
        pltpu.make_async_copy(
            table_hbm.at[pl.ds(0, CHUNK)], wide_ref, sem
        ).wait()

    def compute(wide_ref, pos0):
        def row_body(r, pos):
            for dd in range(D // LANES):
                sl = pl.ds(dd * LANES, LANES)
                comp_v[r, sl] = wide_ref[r, sl] + pe_v[pl.ds(pos * D + dd * LANES, LANES)]
            nxt = pos + 1
            return lax.select(nxt == L, 0, nxt)

        return lax.fori_loop(0, CHUNK, row_body, pos0, unroll=8)

    def step(c, nidx_ref, noff, wide, nwide, sem, nsem, pos):
        issue_gather(nidx_ref, noff, nwide, nsem)
        wait_gather(wide, sem)
        pos = compute(wide, pos)
        pltpu.sync_copy(comp_v, out_hbm.at[pl.ds(row0 + c * CHUNK, CHUNK)])
        return pos

    issue_gather(idx_a, 0, wide0, sg0)

    def pair_body(p, pos):
        c0 = 2 * BLK * p
        bufs = ((wide0, sg0), (wide1, sg1))
        cp_b = pltpu.async_copy(
            seq_hbm.at[pl.ds(row0 + (2 * p + 1) * IDX_BLK, IDX_BLK)], idx_b, si_b
        )
        cp_a = None
        for k in range(2 * BLK):
            if k == BLK - 1:
                cp_b.wait()
            if k == BLK:
                cp_a = pltpu.async_copy(
                    seq_hbm.at[
                        pl.ds(row0 + lax.min(2 * p + 2, 9) * IDX_BLK, IDX_BLK)
                    ],
                    idx_a,
                    si_a,
                )
            if k == 2 * BLK - 1:
                cp_a.wait()
            kn = k + 1
            nidx_ref = idx_a if (kn // BLK) % 2 == 0 else idx_b
            noff = (kn % BLK) * CHUNK
            wide, sem = bufs[k % 2]
            nwide, nsem = bufs[kn % 2]
            pos = step(c0 + k, nidx_ref, noff, wide, nwide, sem, nsem, pos)
        return pos

    lax.fori_loop(0, N_PAIRS, pair_body, jnp.int32(0))
    wait_gather(wide0, sg0)


def kernel(sequence, table):
    assert sequence.shape == (B, L), sequence.shape
    assert table.shape[1] == D, table.shape
    seq1d = sequence.reshape(-1).astype(jnp.int32)
    table_pad = jnp.pad(table, ((0, 0), (0, DPAD - D)))
    pe = _positional_encoding()
    out = _sc_embed(seq1d, pe, table_pad)
    return out.reshape(B, L, D)


# speedup vs baseline: 1.0023x; 1.0023x over previous
import functools

import numpy as np
import jax
import jax.numpy as jnp
from jax import lax
from jax.experimental import pallas as pl
from jax.experimental.pallas import tpu as pltpu
from jax.experimental.pallas import tpu_sc as plsc

B = 4096
L = 200
D = 64
DPAD = 128
N_ROWS = B * L
NW = 32
PER_W = N_ROWS // NW
CHUNK = 128
N_CHUNKS = PER_W // CHUNK
NBUF = 4
STEPS = 4
N_ITERS = N_CHUNKS // STEPS
LANES = 16


def _positional_encoding():
    pos = np.arange(L, dtype=np.float32)[:, None]
    div = np.exp(np.arange(0, D, 2, dtype=np.float32) * (-np.log(10000.0) / D))
    pe = np.zeros((L, D), dtype=np.float32)
    pe[:, 0::2] = np.sin(pos * div)
    pe[:, 1::2] = np.cos(pos * div)
    return jnp.asarray(pe.reshape(-1))


@functools.partial(
    pl.kernel,
    mesh=plsc.VectorSubcoreMesh(core_axis_name="c", subcore_axis_name="s"),
    compiler_params=pltpu.CompilerParams(use_tc_tiling_on_sc=True),
    out_type=jax.ShapeDtypeStruct((N_ROWS, D), jnp.float32),
    scratch_types=[
        pltpu.VMEM((PER_W,), jnp.int32),
        pltpu.VMEM((CHUNK, DPAD), jnp.float32),
        pltpu.VMEM((CHUNK, DPAD), jnp.float32),
        pltpu.VMEM((CHUNK, DPAD), jnp.float32),
        pltpu.VMEM((CHUNK, DPAD), jnp.float32),
        pltpu.VMEM((CHUNK, D), jnp.float32),
        pltpu.VMEM((L * D,), jnp.float32),
        pltpu.SemaphoreType.DMA,
        pltpu.SemaphoreType.DMA,
        pltpu.SemaphoreType.DMA,
        pltpu.SemaphoreType.DMA,
    ],
)
def _sc_embed(seq_hbm, pe_hbm, table_hbm, out_hbm,
              idx_v, w0, w1, w2, w3, comp_v, pe_v,
              sem0, sem1, sem2, sem3):
    nc = lax.axis_size("c")
    wid = lax.axis_index("s") * nc + lax.axis_index("c")
    row0 = wid * PER_W
    bufs = ((w0, sem0), (w1, sem1), (w2, sem2), (w3, sem3))
    pltpu.sync_copy(pe_hbm, pe_v)
    pltpu.sync_copy(seq_hbm.at[pl.ds(row0, PER_W)], idx_v)

    def issue_gather(c, wide_ref, sem):
        cc = lax.min(c, N_CHUNKS - 1)
        pltpu.async_copy(
            table_hbm.at[idx_v.at[pl.ds(cc * CHUNK, CHUNK)]], wide_ref, sem
        )

    def wait_gather(wide_ref, sem):
        pltpu.make_async_copy(
            table_hbm.at[pl.ds(0, CHUNK)], wide_ref, sem
        ).wait()

    def compute(wide_ref, pos0):
        def row_body(r, pos):
            for dd in range(D // LANES):
                sl = pl.ds(dd * LANES, LANES)
                comp_v[r, sl] = (
                    wide_ref[r, sl] + pe_v[pl.ds(pos * D + dd * LANES, LANES)]
                )
            nxt = pos + 1
            return lax.select(nxt == L, 0, nxt)

        return lax.fori_loop(0, CHUNK, row_body, pos0, unroll=8)

    for k in range(NBUF - 1):
        issue_gather(k, *bufs[k])

    def body(i, pos):
        c0 = STEPS * i
        for k in range(STEPS):
            c = c0 + k
            issue_gather(c + NBUF - 1, *bufs[(k + NBUF - 1) % NBUF])
            wait_gather(*bufs[k])
            pos = compute(bufs[k][0], pos)
            pltpu.sync_copy(
                comp_v, out_hbm.at[pl.ds(row0 + c * CHUNK, CHUNK)]
            )
        return pos

    lax.fori_loop(0, N_ITERS, body, jnp.int32(0))
    for k in range(NBUF - 1):
        wait_gather(*bufs[k])


def kernel(sequence, table):
    assert sequence.shape == (B, L), sequence.shape
    assert table.shape[1] == D, table.shape
    seq1d = sequence.reshape(-1).astype(jnp.int32)
    table_pad = jnp.pad(table, ((0, 0), (0, DPAD - D)))
    pe = _positional_encoding()
    out = _sc_embed(seq1d, pe, table_pad)
    return out.reshape(B, L, D)
